# baseline (speedup 1.0000x reference)
import math

import jax
import jax.numpy as jnp
from jax import lax
from jax.experimental import pallas as pl
from jax.experimental.pallas import tpu as pltpu
from jax.experimental.pallas import tpu_sc as plsc

_N = 20000
_L = 16
_TILE = 128
_WCOLS = 5 * _TILE
_SCOLS = 4 * _TILE
_SPLIT = 29
_SBASE = _SPLIT * _WCOLS
assert _SBASE + 3 * _SCOLS == 157 * _TILE
_NG = _WCOLS // _L

_SCALE_CLAMP = math.log(224.0 / 8.0)
_BG = -1e8


_H1 = 2 * _TILE
_H2L = _WCOLS - _H1
_H2S = _SCOLS - _H1


def _sc_body(d_hbm, a_hbm, o_hbm, d_v, a_v, o_v, sem1, sem2, sem3):
    wid = lax.axis_index("s") * 2 + lax.axis_index("c")
    is_long = wid < _SPLIT
    start = jnp.where(is_long, wid * _WCOLS, _SBASE + (wid - _SPLIT) * _SCOLS)
    start = pl.multiple_of(start, _TILE)

    i1d = pltpu.async_copy(d_hbm.at[:, pl.ds(start, _H1)],
                           d_v.at[:, pl.ds(0, _H1)], sem1)
    i1a = pltpu.async_copy(a_hbm.at[:, pl.ds(start, _H1)],
                           a_v.at[:, pl.ds(0, _H1)], sem1)

    @pl.when(is_long)
    def _():
        pltpu.async_copy(d_hbm.at[:, pl.ds(start + _H1, _H2L)],
                         d_v.at[:, pl.ds(_H1, _H2L)], sem2)
        pltpu.async_copy(a_hbm.at[:, pl.ds(start + _H1, _H2L)],
                         a_v.at[:, pl.ds(_H1, _H2L)], sem2)

    @pl.when(jnp.logical_not(is_long))
    def _():
        pltpu.async_copy(d_hbm.at[:, pl.ds(start + _H1, _H2S)],
                         d_v.at[:, pl.ds(_H1, _H2S)], sem2)
        pltpu.async_copy(a_hbm.at[:, pl.ds(start + _H1, _H2S)],
                         a_v.at[:, pl.ds(_H1, _H2S)], sem2)

    clamp = jnp.full((_L,), _SCALE_CLAMP, jnp.float32)
    bg = jnp.full((_L,), _BG, jnp.float32)

    def _decode_group(g):
        o = g * _L
        dx = d_v[0, pl.ds(o, _L)]
        dy = d_v[1, pl.ds(o, _L)]
        dw = d_v[2, pl.ds(o, _L)]
        dh = d_v[3, pl.ds(o, _L)]
        x1 = a_v[0, pl.ds(o, _L)]
        y1 = a_v[1, pl.ds(o, _L)]
        x2 = a_v[2, pl.ds(o, _L)]
        y2 = a_v[3, pl.ds(o, _L)]
        pw = x2 - x1
        ph = y2 - y1
        px = (x1 + x2) * 0.5
        py = (y1 + y2) * 0.5
        bw2 = jnp.exp(jnp.minimum(dw, clamp)) * pw * 0.5
        bh2 = jnp.exp(jnp.minimum(dh, clamp)) * ph * 0.5
        bx = dx * pw + px
        by = dy * ph + py
        fg = dx != bg
        o_v[0, pl.ds(o, _L)] = jnp.where(fg, bx - bw2, bg)
        o_v[1, pl.ds(o, _L)] = jnp.where(fg, by - bh2, bg)
        o_v[2, pl.ds(o, _L)] = jnp.where(fg, bx + bw2, bg)
        o_v[3, pl.ds(o, _L)] = jnp.where(fg, by + bh2, bg)

    i1d.wait()
    i1a.wait()

    @plsc.parallel_loop(0, _H1 // _L, unroll=8)
    def _step1(g):
        _decode_group(g)

    o1 = pltpu.async_copy(o_v.at[:, pl.ds(0, _H1)],
                          o_hbm.at[:, pl.ds(start, _H1)], sem3)

    @pl.when(is_long)
    def _():
        pltpu.make_async_copy(d_hbm.at[:, pl.ds(start + _H1, _H2L)],
                              d_v.at[:, pl.ds(_H1, _H2L)], sem2).wait()
        pltpu.make_async_copy(a_hbm.at[:, pl.ds(start + _H1, _H2L)],
                              a_v.at[:, pl.ds(_H1, _H2L)], sem2).wait()

    @pl.when(jnp.logical_not(is_long))
    def _():
        pltpu.make_async_copy(d_hbm.at[:, pl.ds(start + _H1, _H2S)],
                              d_v.at[:, pl.ds(_H1, _H2S)], sem2).wait()
        pltpu.make_async_copy(a_hbm.at[:, pl.ds(start + _H1, _H2S)],
                              a_v.at[:, pl.ds(_H1, _H2S)], sem2).wait()

    @plsc.parallel_loop(_H1 // _L, _NG, unroll=8)
    def _step2(g):
        _decode_group(g)

    @pl.when(is_long)
    def _():
        pltpu.sync_copy(o_v.at[:, pl.ds(_H1, _H2L)],
                        o_hbm.at[:, pl.ds(start + _H1, _H2L)])

    @pl.when(jnp.logical_not(is_long))
    def _():
        pltpu.sync_copy(o_v.at[:, pl.ds(_H1, _H2S)],
                        o_hbm.at[:, pl.ds(start + _H1, _H2S)])

    o1.wait()


_decode = pl.kernel(
    _sc_body,
    out_type=jax.ShapeDtypeStruct((4, _N), jnp.float32),
    mesh=plsc.VectorSubcoreMesh(core_axis_name="c", subcore_axis_name="s",
                                num_cores=2, num_subcores=16),
    compiler_params=pltpu.CompilerParams(
        needs_layout_passes=False,
        skip_device_barrier=True,
        disable_bounds_checks=True,
        disable_semaphore_checks=True,
    ),
    scratch_types=[
        pltpu.VMEM((4, _WCOLS), jnp.float32),
        pltpu.VMEM((4, _WCOLS), jnp.float32),
        pltpu.VMEM((4, _WCOLS), jnp.float32),
        pltpu.SemaphoreType.DMA,
        pltpu.SemaphoreType.DMA,
        pltpu.SemaphoreType.DMA,
    ],
)


def kernel(deltas, anchors):
    return _decode(deltas.T, anchors.T).T


# speedup vs baseline: 1.0121x; 1.0121x over previous
import math

import jax
import jax.numpy as jnp
from jax import lax
from jax.experimental import pallas as pl
from jax.experimental.pallas import tpu as pltpu
from jax.experimental.pallas import tpu_sc as plsc

_N = 20000
_L = 16
_TILE = 128
_WCOLS = 5 * _TILE
_SCOLS = 4 * _TILE
_SPLIT = 29
_SBASE = _SPLIT * _WCOLS
assert _SBASE + 3 * _SCOLS == 157 * _TILE
_NG = _WCOLS // _L

_SCALE_CLAMP = math.log(224.0 / 8.0)
_BG = -1e8


def _sc_body(d_hbm, a_hbm, o_hbm, d_v, a_v, o_v, sem):
    wid = lax.axis_index("s") * 2 + lax.axis_index("c")

    @pl.when(wid < _SPLIT)
    def _():
        start = pl.multiple_of(wid * _WCOLS, _TILE)
        c1 = pltpu.async_copy(d_hbm.at[:, pl.ds(start, _WCOLS)], d_v, sem)
        c2 = pltpu.async_copy(a_hbm.at[:, pl.ds(start, _WCOLS)], a_v, sem)
        c1.wait()
        c2.wait()

    @pl.when(wid >= _SPLIT)
    def _():
        start = pl.multiple_of(_SBASE + (wid - _SPLIT) * _SCOLS, _TILE)
        c1 = pltpu.async_copy(d_hbm.at[:, pl.ds(start, _SCOLS)],
                              d_v.at[:, pl.ds(0, _SCOLS)], sem)
        c2 = pltpu.async_copy(a_hbm.at[:, pl.ds(start, _SCOLS)],
                              a_v.at[:, pl.ds(0, _SCOLS)], sem)
        c1.wait()
        c2.wait()

    clamp = jnp.full((_L,), _SCALE_CLAMP, jnp.float32)
    bg = jnp.full((_L,), _BG, jnp.float32)

    @plsc.parallel_loop(0, _NG, unroll=8)
    def _step(g):
        o = g * _L
        dx = d_v[0, pl.ds(o, _L)]
        dy = d_v[1, pl.ds(o, _L)]
        dw = d_v[2, pl.ds(o, _L)]
        dh = d_v[3, pl.ds(o, _L)]
        x1 = a_v[0, pl.ds(o, _L)]
        y1 = a_v[1, pl.ds(o, _L)]
        x2 = a_v[2, pl.ds(o, _L)]
        y2 = a_v[3, pl.ds(o, _L)]
        pw = x2 - x1
        ph = y2 - y1
        px = (x1 + x2) * 0.5
        py = (y1 + y2) * 0.5
        bw2 = jnp.exp(jnp.minimum(dw, clamp)) * pw * 0.5
        bh2 = jnp.exp(jnp.minimum(dh, clamp)) * ph * 0.5
        bx = dx * pw + px
        by = dy * ph + py
        fg = dx != bg
        o_v[0, pl.ds(o, _L)] = jnp.where(fg, bx - bw2, bg)
        o_v[1, pl.ds(o, _L)] = jnp.where(fg, by - bh2, bg)
        o_v[2, pl.ds(o, _L)] = jnp.where(fg, bx + bw2, bg)
        o_v[3, pl.ds(o, _L)] = jnp.where(fg, by + bh2, bg)

    @pl.when(wid < _SPLIT)
    def _():
        start = pl.multiple_of(wid * _WCOLS, _TILE)
        pltpu.sync_copy(o_v, o_hbm.at[:, pl.ds(start, _WCOLS)])

    @pl.when(wid >= _SPLIT)
    def _():
        start = pl.multiple_of(_SBASE + (wid - _SPLIT) * _SCOLS, _TILE)
        pltpu.sync_copy(o_v.at[:, pl.ds(0, _SCOLS)],
                        o_hbm.at[:, pl.ds(start, _SCOLS)])


_decode = pl.kernel(
    _sc_body,
    out_type=jax.ShapeDtypeStruct((4, _N), jnp.float32),
    mesh=plsc.VectorSubcoreMesh(core_axis_name="c", subcore_axis_name="s",
                                num_cores=2, num_subcores=16),
    compiler_params=pltpu.CompilerParams(
        needs_layout_passes=False,
        skip_device_barrier=True,
        disable_bounds_checks=True,
        disable_semaphore_checks=True,
    ),
    scratch_types=[
        pltpu.VMEM((4, _WCOLS), jnp.float32),
        pltpu.VMEM((4, _WCOLS), jnp.float32),
        pltpu.VMEM((4, _WCOLS), jnp.float32),
        pltpu.SemaphoreType.DMA,
    ],
)


def kernel(deltas, anchors):
    return _decode(deltas.T, anchors.T).T


# speedup vs baseline: 1.0185x; 1.0063x over previous
import math

import jax
import jax.numpy as jnp
from jax import lax
from jax.experimental import pallas as pl
from jax.experimental.pallas import tpu as pltpu
from jax.experimental.pallas import tpu_sc as plsc

_N = 20000
_L = 16
_TILE = 128
_WCOLS = 5 * _TILE
_SCOLS = 4 * _TILE
_SPLIT = 29
_SBASE = _SPLIT * _WCOLS
assert _SBASE + 3 * _SCOLS == 157 * _TILE
_NG = _WCOLS // _L

_SCALE_CLAMP = math.log(224.0 / 8.0)
_BG = -1e8


def _sc_body(d_hbm, a_hbm, o_hbm, d_v, a_v, o_v, sem):
    wid = lax.axis_index("s") * 2 + lax.axis_index("c")

    @pl.when(wid < _SPLIT)
    def _():
        start = pl.multiple_of(wid * _WCOLS, _TILE)
        c1 = pltpu.async_copy(d_hbm.at[:, pl.ds(start, _WCOLS)], d_v, sem)
        c2 = pltpu.async_copy(a_hbm.at[:, pl.ds(start, _WCOLS)], a_v, sem)
        c1.wait()
        c2.wait()

    @pl.when(wid >= _SPLIT)
    def _():
        start = pl.multiple_of(_SBASE + (wid - _SPLIT) * _SCOLS, _TILE)
        c1 = pltpu.async_copy(d_hbm.at[:, pl.ds(start, _SCOLS)],
                              d_v.at[:, pl.ds(0, _SCOLS)], sem)
        c2 = pltpu.async_copy(a_hbm.at[:, pl.ds(start, _SCOLS)],
                              a_v.at[:, pl.ds(0, _SCOLS)], sem)
        c1.wait()
        c2.wait()

    clamp = jnp.full((_L,), _SCALE_CLAMP, jnp.float32)
    bg = jnp.full((_L,), _BG, jnp.float32)

    @plsc.parallel_loop(0, _NG, unroll=4)
    def _step(g):
        o = g * _L
        dx = d_v[0, pl.ds(o, _L)]
        dy = d_v[1, pl.ds(o, _L)]
        dw = d_v[2, pl.ds(o, _L)]
        dh = d_v[3, pl.ds(o, _L)]
        x1 = a_v[0, pl.ds(o, _L)]
        y1 = a_v[1, pl.ds(o, _L)]
        x2 = a_v[2, pl.ds(o, _L)]
        y2 = a_v[3, pl.ds(o, _L)]
        pw = x2 - x1
        ph = y2 - y1
        px = (x1 + x2) * 0.5
        py = (y1 + y2) * 0.5
        bw2 = jnp.exp(jnp.minimum(dw, clamp)) * pw * 0.5
        bh2 = jnp.exp(jnp.minimum(dh, clamp)) * ph * 0.5
        bx = dx * pw + px
        by = dy * ph + py
        fg = dx != bg
        o_v[0, pl.ds(o, _L)] = jnp.where(fg, bx - bw2, bg)
        o_v[1, pl.ds(o, _L)] = jnp.where(fg, by - bh2, bg)
        o_v[2, pl.ds(o, _L)] = jnp.where(fg, bx + bw2, bg)
        o_v[3, pl.ds(o, _L)] = jnp.where(fg, by + bh2, bg)

    @pl.when(wid < _SPLIT)
    def _():
        start = pl.multiple_of(wid * _WCOLS, _TILE)
        pltpu.sync_copy(o_v, o_hbm.at[:, pl.ds(start, _WCOLS)])

    @pl.when(wid >= _SPLIT)
    def _():
        start = pl.multiple_of(_SBASE + (wid - _SPLIT) * _SCOLS, _TILE)
        pltpu.sync_copy(o_v.at[:, pl.ds(0, _SCOLS)],
                        o_hbm.at[:, pl.ds(start, _SCOLS)])


_decode = pl.kernel(
    _sc_body,
    out_type=jax.ShapeDtypeStruct((4, _N), jnp.float32),
    mesh=plsc.VectorSubcoreMesh(core_axis_name="c", subcore_axis_name="s",
                                num_cores=2, num_subcores=16),
    compiler_params=pltpu.CompilerParams(
        needs_layout_passes=False,
        skip_device_barrier=True,
        disable_bounds_checks=True,
        disable_semaphore_checks=True,
    ),
    scratch_types=[
        pltpu.VMEM((4, _WCOLS), jnp.float32),
        pltpu.VMEM((4, _WCOLS), jnp.float32),
        pltpu.VMEM((4, _WCOLS), jnp.float32),
        pltpu.SemaphoreType.DMA,
    ],
)


def kernel(deltas, anchors):
    return _decode(deltas.T, anchors.T).T


# speedup vs baseline: 1.0261x; 1.0075x over previous
import math

import jax
import jax.numpy as jnp
from jax import lax
from jax.experimental import pallas as pl
from jax.experimental.pallas import tpu as pltpu
from jax.experimental.pallas import tpu_sc as plsc

_N = 20000
_L = 16
_TILE = 128
_WCOLS = 5 * _TILE
_SCOLS = 4 * _TILE
_SPLIT = 29
_SBASE = _SPLIT * _WCOLS
assert _SBASE + 3 * _SCOLS == 157 * _TILE
_NG = _WCOLS // _L

_SCALE_CLAMP = math.log(224.0 / 8.0)
_BG = -1e8


def _sc_body(d_hbm, a_hbm, o_hbm, d_v, a_v, o_v, sem):
    wid = lax.axis_index("s") * 2 + lax.axis_index("c")

    @pl.when(wid < _SPLIT)
    def _():
        start = pl.multiple_of(wid * _WCOLS, _TILE)
        c1 = pltpu.async_copy(d_hbm.at[:, pl.ds(start, _WCOLS)], d_v, sem)
        c2 = pltpu.async_copy(a_hbm.at[:, pl.ds(start, _WCOLS)], a_v, sem)
        c1.wait()
        c2.wait()

    @pl.when(wid >= _SPLIT)
    def _():
        start = pl.multiple_of(_SBASE + (wid - _SPLIT) * _SCOLS, _TILE)
        c1 = pltpu.async_copy(d_hbm.at[:, pl.ds(start, _SCOLS)],
                              d_v.at[:, pl.ds(0, _SCOLS)], sem)
        c2 = pltpu.async_copy(a_hbm.at[:, pl.ds(start, _SCOLS)],
                              a_v.at[:, pl.ds(0, _SCOLS)], sem)
        c1.wait()
        c2.wait()

    clamp = jnp.full((_L,), _SCALE_CLAMP, jnp.float32)
    bg = jnp.full((_L,), _BG, jnp.float32)

    @plsc.parallel_loop(0, _NG, unroll=2)
    def _step(g):
        o = g * _L
        dx = d_v[0, pl.ds(o, _L)]
        dy = d_v[1, pl.ds(o, _L)]
        dw = d_v[2, pl.ds(o, _L)]
        dh = d_v[3, pl.ds(o, _L)]
        x1 = a_v[0, pl.ds(o, _L)]
        y1 = a_v[1, pl.ds(o, _L)]
        x2 = a_v[2, pl.ds(o, _L)]
        y2 = a_v[3, pl.ds(o, _L)]
        pw = x2 - x1
        ph = y2 - y1
        px = (x1 + x2) * 0.5
        py = (y1 + y2) * 0.5
        bw2 = jnp.exp(jnp.minimum(dw, clamp)) * pw * 0.5
        bh2 = jnp.exp(jnp.minimum(dh, clamp)) * ph * 0.5
        bx = dx * pw + px
        by = dy * ph + py
        fg = dx != bg
        o_v[0, pl.ds(o, _L)] = jnp.where(fg, bx - bw2, bg)
        o_v[1, pl.ds(o, _L)] = jnp.where(fg, by - bh2, bg)
        o_v[2, pl.ds(o, _L)] = jnp.where(fg, bx + bw2, bg)
        o_v[3, pl.ds(o, _L)] = jnp.where(fg, by + bh2, bg)

    @pl.when(wid < _SPLIT)
    def _():
        start = pl.multiple_of(wid * _WCOLS, _TILE)
        pltpu.sync_copy(o_v, o_hbm.at[:, pl.ds(start, _WCOLS)])

    @pl.when(wid >= _SPLIT)
    def _():
        start = pl.multiple_of(_SBASE + (wid - _SPLIT) * _SCOLS, _TILE)
        pltpu.sync_copy(o_v.at[:, pl.ds(0, _SCOLS)],
                        o_hbm.at[:, pl.ds(start, _SCOLS)])


_decode = pl.kernel(
    _sc_body,
    out_type=jax.ShapeDtypeStruct((4, _N), jnp.float32),
    mesh=plsc.VectorSubcoreMesh(core_axis_name="c", subcore_axis_name="s",
                                num_cores=2, num_subcores=16),
    compiler_params=pltpu.CompilerParams(
        needs_layout_passes=False,
        skip_device_barrier=True,
        disable_bounds_checks=True,
        disable_semaphore_checks=True,
    ),
    scratch_types=[
        pltpu.VMEM((4, _WCOLS), jnp.float32),
        pltpu.VMEM((4, _WCOLS), jnp.float32),
        pltpu.VMEM((4, _WCOLS), jnp.float32),
        pltpu.SemaphoreType.DMA,
    ],
)


def kernel(deltas, anchors):
    return _decode(deltas.T, anchors.T).T


# speedup vs baseline: 1.0379x; 1.0115x over previous
import math

import jax
import jax.numpy as jnp
from jax import lax
from jax.experimental import pallas as pl
from jax.experimental.pallas import tpu as pltpu
from jax.experimental.pallas import tpu_sc as plsc

_N = 20000
_L = 16
_TILE = 128
_WCOLS = 5 * _TILE
_NTILES = 157
_CLAMP = _NTILES * _TILE - _WCOLS
assert 31 * _WCOLS >= _CLAMP
_NG = _WCOLS // _L

_SCALE_CLAMP = math.log(224.0 / 8.0)
_BG = -1e8


def _sc_body(d_hbm, a_hbm, o_hbm, d_v, a_v, o_v, sem):
    wid = lax.axis_index("s") * 2 + lax.axis_index("c")
    start = pl.multiple_of(jnp.minimum(wid * _WCOLS, _CLAMP), _TILE)
    c1 = pltpu.async_copy(d_hbm.at[:, pl.ds(start, _WCOLS)], d_v, sem)
    c2 = pltpu.async_copy(a_hbm.at[:, pl.ds(start, _WCOLS)], a_v, sem)
    c1.wait()
    c2.wait()

    clamp = jnp.full((_L,), _SCALE_CLAMP, jnp.float32)
    bg = jnp.full((_L,), _BG, jnp.float32)

    @plsc.parallel_loop(0, _NG, unroll=2)
    def _step(g):
        o = g * _L
        dx = d_v[0, pl.ds(o, _L)]
        dy = d_v[1, pl.ds(o, _L)]
        dw = d_v[2, pl.ds(o, _L)]
        dh = d_v[3, pl.ds(o, _L)]
        x1 = a_v[0, pl.ds(o, _L)]
        y1 = a_v[1, pl.ds(o, _L)]
        x2 = a_v[2, pl.ds(o, _L)]
        y2 = a_v[3, pl.ds(o, _L)]
        pw = x2 - x1
        ph = y2 - y1
        px = (x1 + x2) * 0.5
        py = (y1 + y2) * 0.5
        bw2 = jnp.exp(jnp.minimum(dw, clamp)) * pw * 0.5
        bh2 = jnp.exp(jnp.minimum(dh, clamp)) * ph * 0.5
        bx = dx * pw + px
        by = dy * ph + py
        fg = dx != bg
        o_v[0, pl.ds(o, _L)] = jnp.where(fg, bx - bw2, bg)
        o_v[1, pl.ds(o, _L)] = jnp.where(fg, by - bh2, bg)
        o_v[2, pl.ds(o, _L)] = jnp.where(fg, bx + bw2, bg)
        o_v[3, pl.ds(o, _L)] = jnp.where(fg, by + bh2, bg)

    pltpu.sync_copy(o_v, o_hbm.at[:, pl.ds(start, _WCOLS)])


_decode = pl.kernel(
    _sc_body,
    out_type=jax.ShapeDtypeStruct((4, _N), jnp.float32),
    mesh=plsc.VectorSubcoreMesh(core_axis_name="c", subcore_axis_name="s",
                                num_cores=2, num_subcores=16),
    compiler_params=pltpu.CompilerParams(
        needs_layout_passes=False,
        skip_device_barrier=True,
        disable_bounds_checks=True,
        disable_semaphore_checks=True,
    ),
    scratch_types=[
        pltpu.VMEM((4, _WCOLS), jnp.float32),
        pltpu.VMEM((4, _WCOLS), jnp.float32),
        pltpu.VMEM((4, _WCOLS), jnp.float32),
        pltpu.SemaphoreType.DMA,
    ],
)


def kernel(deltas, anchors):
    return _decode(deltas.T, anchors.T).T
